# Initial kernel scaffold; baseline (speedup 1.0000x reference)
import functools

import jax
import jax.numpy as jnp
from jax import lax
from jax.experimental import pallas as pl
from jax.experimental.pallas import tpu as pltpu
from jax.experimental.pallas import tpu_sc as plsc

_G = 128


@functools.lru_cache(maxsize=None)
def _make_gather(vocab_rows: int, d: int, n: int):
  info = plsc.get_sparse_core_info()
  nw = info.num_cores * info.num_subcores
  assert n % (nw * _G) == 0
  ngw = n // (nw * _G)

  mesh = plsc.VectorSubcoreMesh(core_axis_name="c", subcore_axis_name="s")

  @functools.partial(
      pl.kernel,
      out_type=jax.ShapeDtypeStruct((n, d), jnp.float32),
      mesh=mesh,
      scratch_types=[
          pltpu.VMEM((ngw, _G), jnp.int32),
          pltpu.VMEM((2, _G, d), jnp.float32),
          pltpu.SemaphoreType.DMA,
          pltpu.SemaphoreType.DMA,
      ],
  )
  def gather_kernel(table_hbm, idx_hbm, out_hbm, idx_v, rows_v, sem0, sem1):
    wid = lax.axis_index("s") * info.num_cores + lax.axis_index("c")
    rbase = wid * ngw
    obase = rbase * _G
    sems = (sem0, sem1)

    pltpu.sync_copy(idx_hbm.at[pl.ds(rbase, ngw)], idx_v)
    pltpu.async_copy(table_hbm.at[idx_v.at[0]], rows_v.at[0], sems[0])

    @pl.loop(0, ngw, step=2)
    def _step(j):
      for b in range(2):
        i = j + b
        @pl.when(i + 1 < ngw)
        def _():
          pltpu.async_copy(
              table_hbm.at[idx_v.at[i + 1]], rows_v.at[1 - b], sems[1 - b]
          )
        pltpu.make_async_copy(
            table_hbm.at[idx_v.at[i]], rows_v.at[b], sems[b]
        ).wait()
        pltpu.sync_copy(rows_v.at[b], out_hbm.at[pl.ds(obase + i * _G, _G)])

  return gather_kernel


def kernel(W_in, data):
  b, s = data.shape
  n = b * s
  idx = data.reshape(-1).astype(jnp.int32).reshape(n // _G, _G)
  out = _make_gather(W_in.shape[0], W_in.shape[1], n)(W_in, idx)
  return out.reshape(b, s, W_in.shape[1])


# speedup vs baseline: 1.8400x; 1.8400x over previous
import functools

import jax
import jax.numpy as jnp
from jax import lax
from jax.experimental import pallas as pl
from jax.experimental.pallas import tpu as pltpu
from jax.experimental.pallas import tpu_sc as plsc

_G = 128


@functools.lru_cache(maxsize=None)
def _make_gather(vocab_rows: int, d: int, n: int):
  info = plsc.get_sparse_core_info()
  nw = info.num_cores * info.num_subcores
  assert n % (nw * _G) == 0
  ngw = n // (nw * _G)

  mesh = plsc.VectorSubcoreMesh(core_axis_name="c", subcore_axis_name="s")

  @functools.partial(
      pl.kernel,
      out_type=jax.ShapeDtypeStruct((n, d), jnp.float32),
      mesh=mesh,
      compiler_params=pltpu.CompilerParams(use_tc_tiling_on_sc=False),
      scratch_types=[
          pltpu.VMEM((ngw, _G), jnp.int32),
          pltpu.VMEM((2, _G, d), jnp.float32),
          pltpu.SemaphoreType.DMA,
          pltpu.SemaphoreType.DMA,
      ],
  )
  def gather_kernel(table_hbm, idx_hbm, out_hbm, idx_v, rows_v, sem0, sem1):
    wid = lax.axis_index("s") * info.num_cores + lax.axis_index("c")
    rbase = wid * ngw
    obase = rbase * _G
    sems = (sem0, sem1)

    pltpu.sync_copy(idx_hbm.at[pl.ds(rbase, ngw)], idx_v)
    pltpu.async_copy(table_hbm.at[idx_v.at[0]], rows_v.at[0], sems[0])

    @pl.loop(0, ngw, step=2)
    def _step(j):
      for b in range(2):
        i = j + b
        @pl.when(i + 1 < ngw)
        def _():
          pltpu.async_copy(
              table_hbm.at[idx_v.at[i + 1]], rows_v.at[1 - b], sems[1 - b]
          )
        pltpu.make_async_copy(
            table_hbm.at[idx_v.at[i]], rows_v.at[b], sems[b]
        ).wait()
        pltpu.sync_copy(rows_v.at[b], out_hbm.at[pl.ds(obase + i * _G, _G)])

  return gather_kernel


def kernel(W_in, data):
  b, s = data.shape
  n = b * s
  idx = data.reshape(-1).astype(jnp.int32).reshape(n // _G, _G)
  out = _make_gather(W_in.shape[0], W_in.shape[1], n)(W_in, idx)
  return out.reshape(b, s, W_in.shape[1])


# speedup vs baseline: 1.8784x; 1.0209x over previous
import functools

import jax
import jax.numpy as jnp
from jax import lax
from jax.experimental import pallas as pl
from jax.experimental.pallas import tpu as pltpu
from jax.experimental.pallas import tpu_sc as plsc

_G = 128
_NBUF = 8


@functools.lru_cache(maxsize=None)
def _make_gather(vocab_rows: int, d: int, n: int):
  info = plsc.get_sparse_core_info()
  nw = info.num_cores * info.num_subcores
  assert n % (nw * _G * _NBUF) == 0
  ngw = n // (nw * _G)

  mesh = plsc.VectorSubcoreMesh(core_axis_name="c", subcore_axis_name="s")

  @functools.partial(
      pl.kernel,
      out_type=jax.ShapeDtypeStruct((n, d), jnp.float32),
      mesh=mesh,
      compiler_params=pltpu.CompilerParams(use_tc_tiling_on_sc=False),
      scratch_types=[
          pltpu.VMEM((ngw, _G), jnp.int32),
          pltpu.VMEM((_NBUF, _G, d), jnp.float32),
          pltpu.SemaphoreType.DMA((_NBUF,)),
          pltpu.SemaphoreType.DMA((_NBUF,)),
      ],
  )
  def gather_kernel(table_hbm, idx_hbm, out_hbm, idx_v, rows_v, gsem, wsem):
    wid = lax.axis_index("s") * info.num_cores + lax.axis_index("c")
    rbase = wid * ngw
    obase = rbase * _G

    pltpu.sync_copy(idx_hbm.at[pl.ds(rbase, ngw)], idx_v)
    for b in range(_NBUF - 1):
      pltpu.async_copy(table_hbm.at[idx_v.at[b]], rows_v.at[b], gsem.at[b])

    @pl.loop(0, ngw, step=_NBUF)
    def _step(j):
      for b in range(_NBUF):
        i = j + b
        nxt = i + _NBUF - 1
        pslot = (b - 1) % _NBUF

        @pl.when(nxt < ngw)
        def _():
          @pl.when(i >= 1)
          def _():
            pltpu.make_async_copy(
                rows_v.at[pslot],
                out_hbm.at[pl.ds(obase, _G)],
                wsem.at[pslot],
            ).wait()
          pltpu.async_copy(
              table_hbm.at[idx_v.at[nxt]], rows_v.at[pslot], gsem.at[pslot]
          )

        pltpu.make_async_copy(
            table_hbm.at[idx_v.at[i]], rows_v.at[b], gsem.at[b]
        ).wait()
        pltpu.async_copy(
            rows_v.at[b], out_hbm.at[pl.ds(obase + i * _G, _G)], wsem.at[b]
        )

    for b in range(_NBUF):
      pltpu.make_async_copy(
          rows_v.at[b], out_hbm.at[pl.ds(obase, _G)], wsem.at[b]
      ).wait()

  return gather_kernel


def kernel(W_in, data):
  b, s = data.shape
  n = b * s
  idx = data.reshape(-1).astype(jnp.int32).reshape(n // _G, _G)
  out = _make_gather(W_in.shape[0], W_in.shape[1], n)(W_in, idx)
  return out.reshape(b, s, W_in.shape[1])


# speedup vs baseline: 2.8089x; 1.4954x over previous
import functools

import jax
import jax.numpy as jnp
from jax import lax
from jax.experimental import pallas as pl
from jax.experimental.pallas import tpu as pltpu
from jax.experimental.pallas import tpu_sc as plsc

_G = 128
_NBUF = 8


@functools.lru_cache(maxsize=None)
def _make_gather(vocab_rows: int, d: int, n: int):
  info = plsc.get_sparse_core_info()
  nw = info.num_cores * info.num_subcores
  assert n % (nw * _G * _NBUF) == 0
  ngw = n // (nw * _G)

  mesh = plsc.VectorSubcoreMesh(core_axis_name="c", subcore_axis_name="s")

  @functools.partial(
      pl.kernel,
      out_type=jax.ShapeDtypeStruct((n, d), jnp.float32),
      mesh=mesh,
      compiler_params=pltpu.CompilerParams(use_tc_tiling_on_sc=False),
      scratch_types=[
          pltpu.VMEM((ngw, _G), jnp.int32),
          pltpu.VMEM((_NBUF, _G, d), jnp.float32),
          pltpu.SemaphoreType.DMA((_NBUF,)),
          pltpu.SemaphoreType.DMA((_NBUF,)),
      ],
  )
  def gather_kernel(table_hbm, idx_hbm, out_hbm, idx_v, rows_v, gsem, wsem):
    wid = lax.axis_index("s") * info.num_cores + lax.axis_index("c")
    rbase = wid * ngw
    obase = rbase * _G

    pltpu.sync_copy(idx_hbm.at[pl.ds(rbase, ngw)], idx_v)
    for b in range(_NBUF - 1):
      pltpu.async_copy(table_hbm.at[idx_v.at[b]], rows_v.at[b], gsem.at[b])

    @pl.loop(0, ngw, step=_NBUF)
    def _step(j):
      for b in range(_NBUF):
        i = j + b
        nxt = i + _NBUF - 1
        pslot = (b - 1) % _NBUF

        @pl.when(nxt < ngw)
        def _():
          @pl.when(i >= 1)
          def _():
            pltpu.make_async_copy(
                rows_v.at[pslot],
                out_hbm.at[pl.ds(obase, _G)],
                wsem.at[pslot],
            ).wait()
          pltpu.async_copy(
              table_hbm.at[idx_v.at[nxt]], rows_v.at[pslot], gsem.at[pslot]
          )

        pltpu.make_async_copy(
            table_hbm.at[idx_v.at[i]], rows_v.at[b], gsem.at[b]
        ).wait()
        pltpu.async_copy(
            rows_v.at[b], out_hbm.at[pl.ds(obase + i * _G, _G)], wsem.at[b]
        )

    for b in range(_NBUF):
      pltpu.make_async_copy(
          rows_v.at[b], out_hbm.at[pl.ds(obase, _G)], wsem.at[b]
      ).wait()

  return gather_kernel


def kernel(W_in, data):
  from jax.experimental import layout as jlayout

  b, s = data.shape
  n = b * s
  W_rm = jlayout.with_layout_constraint(
      W_in, jlayout.Layout((0, 1), tiling=((8,), (1024,)))
  )
  idx = data.reshape(-1).astype(jnp.int32).reshape(n // _G, _G)
  out = _make_gather(W_in.shape[0], W_in.shape[1], n)(W_rm, idx)
  out3 = out.reshape(b, s, W_in.shape[1])
  return jlayout.with_layout_constraint(
      out3, jlayout.Layout((0, 1, 2), tiling=((8,), (1024,)))
  )
